# baseline (speedup 1.0000x reference)
import functools

import jax
import jax.numpy as jnp
from jax import lax
from jax.experimental import pallas as pl
from jax.experimental.pallas import tpu as pltpu
from jax.experimental.pallas import tpu_sc as plsc

N_COARSE = 12500
N_C_PAD = 12800
RB = 6400
N_FINE = 25000
VAL_DIM = 256
NR_FILTERS = 256
EXT = 9
GROUPS = 32
CG = VAL_DIM // GROUPS
EPS = 1e-5

NC, NS = 2, 16
NW = NC * NS
F_PAD = 25600
ROWS_W = F_PAD // NW
CV = 8
GSZ = CV * EXT
CHUNKS = ROWS_W // CV
NBUF = 2


def _tc_body(x_ref, g_ref, b_ref, w_ref, out_ref, a_ref):
    e = pl.program_id(0)
    i = pl.program_id(1)

    @pl.when((e == 0) & (i == 0))
    def _():
        x = x_ref[...]
        s = jnp.sum(x, axis=0, keepdims=True)
        ss = jnp.sum(x * x, axis=0, keepdims=True)
        ii = lax.broadcasted_iota(jnp.int32, (VAL_DIM, VAL_DIM), 0) // CG
        jj = lax.broadcasted_iota(jnp.int32, (VAL_DIM, VAL_DIM), 1) // CG
        gmat = (ii == jj).astype(jnp.float32)
        n = float(N_COARSE * CG)
        mean = jnp.dot(s, gmat, preferred_element_type=jnp.float32) / n
        ex2 = jnp.dot(ss, gmat, preferred_element_type=jnp.float32) / n
        var = ex2 - mean * mean
        rstd = lax.rsqrt(var + EPS)
        xn = (x - mean) * rstd * g_ref[...] + b_ref[...]
        a_ref[...] = jnp.maximum(xn, 0.0)

    a = a_ref[pl.ds(i * RB, RB), :]
    out_ref[0] = jnp.dot(a, w_ref[0], preferred_element_type=jnp.float32)


def _tc_stage(lv, gamma, beta, w3):
    return pl.pallas_call(
        _tc_body,
        grid=(EXT, N_C_PAD // RB),
        in_specs=[
            pl.BlockSpec((N_C_PAD, VAL_DIM), lambda e, i: (0, 0)),
            pl.BlockSpec((1, VAL_DIM), lambda e, i: (0, 0)),
            pl.BlockSpec((1, VAL_DIM), lambda e, i: (0, 0)),
            pl.BlockSpec((1, VAL_DIM, NR_FILTERS), lambda e, i: (e, 0, 0)),
        ],
        out_specs=pl.BlockSpec((1, RB, NR_FILTERS), lambda e, i: (e, i, 0)),
        out_shape=jax.ShapeDtypeStruct((EXT, N_C_PAD, NR_FILTERS), jnp.float32),
        scratch_shapes=[pltpu.VMEM((N_C_PAD, VAL_DIM), jnp.float32)],
    )(lv, gamma, beta, w3)


@functools.lru_cache(maxsize=None)
def _sc_gather_kernel():
    @functools.partial(
        pl.kernel,
        out_type=jax.ShapeDtypeStruct((F_PAD, NR_FILTERS), jnp.float32),
        mesh=plsc.VectorSubcoreMesh(core_axis_name="c", subcore_axis_name="s"),
        scratch_types=[
            pltpu.VMEM((CHUNKS, GSZ), jnp.int32),
            pltpu.VMEM((GSZ, NR_FILTERS), jnp.float32),
            pltpu.VMEM((GSZ, NR_FILTERS), jnp.float32),
            pltpu.VMEM((CV, NR_FILTERS), jnp.float32),
            pltpu.SemaphoreType.DMA,
            pltpu.SemaphoreType.DMA,
        ],
    )
    def _sc_gather(y_hbm, idx_hbm, out_hbm, idxv, buf0, buf1, ob, sem0, sem1):
        w = lax.axis_index("s") * NC + lax.axis_index("c")
        pltpu.sync_copy(idx_hbm.at[w], idxv)
        obase = w * ROWS_W

        def accum(buf, c):
            def vrow(v, _):
                r0 = v * EXT
                for j in range(NR_FILTERS // 16):
                    sl = pl.ds(j * 16, 16)
                    t0 = buf[r0 + 0, sl] + buf[r0 + 1, sl]
                    t1 = buf[r0 + 2, sl] + buf[r0 + 3, sl]
                    t2 = buf[r0 + 4, sl] + buf[r0 + 5, sl]
                    t3 = buf[r0 + 6, sl] + buf[r0 + 7, sl]
                    ob[v, sl] = ((t0 + t1) + (t2 + t3)) + buf[r0 + 8, sl]
                return 0

            lax.fori_loop(0, CV, vrow, 0)
            pltpu.sync_copy(ob, out_hbm.at[pl.ds(obase + c * CV, CV)])

        pltpu.async_copy(y_hbm.at[idxv.at[0]], buf0, sem0)

        def pair(p, _):
            c0 = 2 * p
            pltpu.async_copy(y_hbm.at[idxv.at[c0 + 1]], buf1, sem1)
            pltpu.make_async_copy(y_hbm.at[idxv.at[c0]], buf0, sem0).wait()
            accum(buf0, c0)

            @pl.when(p + 1 < CHUNKS // 2)
            def _():
                pltpu.async_copy(y_hbm.at[idxv.at[c0 + 2]], buf0, sem0)

            pltpu.make_async_copy(y_hbm.at[idxv.at[c0 + 1]], buf1, sem1).wait()
            accum(buf1, c0 + 1)
            return 0

        lax.fori_loop(0, CHUNKS // 2, pair, 0)

    return _sc_gather


def kernel(lv_coarse, neighbor_idx, gn_gamma, gn_beta, weight):
    w3 = weight.reshape(EXT, VAL_DIM, NR_FILTERS)
    lv_pad = jnp.pad(lv_coarse, ((0, N_C_PAD - N_COARSE), (0, 0)))
    y = _tc_stage(lv_pad, gn_gamma.reshape(1, -1), gn_beta.reshape(1, -1), w3)
    yflat = y.reshape(EXT * N_C_PAD, NR_FILTERS)

    offs = (jnp.arange(EXT, dtype=jnp.int32) * N_C_PAD)[None, :]
    idx = neighbor_idx + offs
    idx = jnp.pad(idx, ((0, F_PAD - N_FINE), (0, 0)))
    idx_t = idx.reshape(NW, CHUNKS, GSZ)

    out = _sc_gather_kernel()(yflat, idx_t)
    return out[:N_FINE]


# speedup vs baseline: 1.5353x; 1.5353x over previous
import functools

import jax
import jax.numpy as jnp
from jax import lax
from jax.experimental import pallas as pl
from jax.experimental.pallas import tpu as pltpu
from jax.experimental.pallas import tpu_sc as plsc

N_COARSE = 12500
N_C_PAD = 12800
RB = 6400
N_FINE = 25000
VAL_DIM = 256
NR_FILTERS = 256
EXT = 9
GROUPS = 32
CG = VAL_DIM // GROUPS
EPS = 1e-5

NC, NS = 2, 16
NW = NC * NS
F_PAD = 25088
ROWS_W = F_PAD // NW
CV = 8
GSZ = CV * EXT
CHUNKS = ROWS_W // CV
NBUF = 2


def _tc_body(x_ref, g_ref, b_ref, w_ref, out_ref, a_ref):
    e = pl.program_id(0)
    i = pl.program_id(1)

    @pl.when((e == 0) & (i == 0))
    def _():
        x = x_ref[...]
        s = jnp.sum(x, axis=0, keepdims=True)
        ss = jnp.sum(x * x, axis=0, keepdims=True)
        ii = lax.broadcasted_iota(jnp.int32, (VAL_DIM, VAL_DIM), 0) // CG
        jj = lax.broadcasted_iota(jnp.int32, (VAL_DIM, VAL_DIM), 1) // CG
        gmat = (ii == jj).astype(jnp.float32)
        n = float(N_COARSE * CG)
        mean = jnp.dot(s, gmat, preferred_element_type=jnp.float32) / n
        ex2 = jnp.dot(ss, gmat, preferred_element_type=jnp.float32) / n
        var = ex2 - mean * mean
        rstd = lax.rsqrt(var + EPS)
        xn = (x - mean) * rstd * g_ref[...] + b_ref[...]
        a_ref[...] = jnp.maximum(xn, 0.0)

    a = a_ref[pl.ds(i * RB, RB), :]
    out_ref[0] = jnp.dot(a, w_ref[0], preferred_element_type=jnp.float32)


def _tc_stage(lv, gamma, beta, w3):
    return pl.pallas_call(
        _tc_body,
        grid=(EXT, N_C_PAD // RB),
        in_specs=[
            pl.BlockSpec((N_C_PAD, VAL_DIM), lambda e, i: (0, 0)),
            pl.BlockSpec((1, VAL_DIM), lambda e, i: (0, 0)),
            pl.BlockSpec((1, VAL_DIM), lambda e, i: (0, 0)),
            pl.BlockSpec((1, VAL_DIM, NR_FILTERS), lambda e, i: (e, 0, 0)),
        ],
        out_specs=pl.BlockSpec((1, RB, NR_FILTERS), lambda e, i: (e, i, 0)),
        out_shape=jax.ShapeDtypeStruct((EXT, N_C_PAD, NR_FILTERS), jnp.float32),
        scratch_shapes=[pltpu.VMEM((N_C_PAD, VAL_DIM), jnp.float32)],
    )(lv, gamma, beta, w3)


@functools.lru_cache(maxsize=None)
def _sc_gather_kernel():
    @functools.partial(
        pl.kernel,
        out_type=jax.ShapeDtypeStruct((F_PAD, NR_FILTERS), jnp.float32),
        mesh=plsc.VectorSubcoreMesh(core_axis_name="c", subcore_axis_name="s"),
        scratch_types=[
            pltpu.VMEM((CHUNKS, GSZ), jnp.int32),
            pltpu.VMEM((GSZ, NR_FILTERS), jnp.float32),
            pltpu.VMEM((GSZ, NR_FILTERS), jnp.float32),
            pltpu.VMEM((CV, NR_FILTERS), jnp.float32),
            pltpu.SemaphoreType.DMA,
            pltpu.SemaphoreType.DMA,
        ],
    )
    def _sc_gather(y_hbm, idx_hbm, out_hbm, idxv, buf0, buf1, ob, sem0, sem1):
        w = lax.axis_index("s") * NC + lax.axis_index("c")
        pltpu.sync_copy(idx_hbm.at[w], idxv)
        obase = w * ROWS_W

        def accum(buf, c):
            def vrow(v, _):
                r0 = v * EXT
                for j in range(NR_FILTERS // 16):
                    sl = pl.ds(j * 16, 16)
                    t0 = buf[r0 + 0, sl] + buf[r0 + 1, sl]
                    t1 = buf[r0 + 2, sl] + buf[r0 + 3, sl]
                    t2 = buf[r0 + 4, sl] + buf[r0 + 5, sl]
                    t3 = buf[r0 + 6, sl] + buf[r0 + 7, sl]
                    ob[v, sl] = ((t0 + t1) + (t2 + t3)) + buf[r0 + 8, sl]
                return 0

            lax.fori_loop(0, CV, vrow, 0)
            pltpu.sync_copy(ob, out_hbm.at[pl.ds(obase + c * CV, CV)])

        pltpu.async_copy(y_hbm.at[idxv.at[0]], buf0, sem0)

        def pair(p, _):
            c0 = 2 * p
            pltpu.async_copy(y_hbm.at[idxv.at[c0 + 1]], buf1, sem1)
            pltpu.make_async_copy(y_hbm.at[idxv.at[c0]], buf0, sem0).wait()
            accum(buf0, c0)

            @pl.when(p + 1 < CHUNKS // 2)
            def _():
                pltpu.async_copy(y_hbm.at[idxv.at[c0 + 2]], buf0, sem0)

            pltpu.make_async_copy(y_hbm.at[idxv.at[c0 + 1]], buf1, sem1).wait()
            accum(buf1, c0 + 1)
            return 0

        lax.fori_loop(0, CHUNKS // 2, pair, 0)

    return _sc_gather


def kernel(lv_coarse, neighbor_idx, gn_gamma, gn_beta, weight):
    w3 = weight.reshape(EXT, VAL_DIM, NR_FILTERS)
    lv_pad = jnp.pad(lv_coarse, ((0, N_C_PAD - N_COARSE), (0, 0)))
    y = _tc_stage(lv_pad, gn_gamma.reshape(1, -1), gn_beta.reshape(1, -1), w3)
    yflat = y.reshape(EXT * N_C_PAD, NR_FILTERS)

    offs = (jnp.arange(EXT, dtype=jnp.int32) * N_C_PAD)[None, :]
    idx = neighbor_idx + offs
    idx = jnp.pad(idx, ((0, F_PAD - N_FINE), (0, 0)))
    idx_t = idx.reshape(NW, CHUNKS, GSZ)

    out = _sc_gather_kernel()(yflat, idx_t)
    return out[:N_FINE]
